# baseline (speedup 1.0000x reference)
import functools

import jax
import jax.numpy as jnp
from jax import lax
from jax.experimental import pallas as pl
from jax.experimental.pallas import tpu as pltpu
from jax.experimental.pallas import tpu_sc as plsc

NUM_HEADS = 16
NUM_TYPES = 32
S = 2048
N = S * S

NC = 2
NS = 16
L = 16
NW = NC * NS
TROWS = S // 8
TROWS_W = TROWS // NW
CW = 512
CHUNK = 8 * CW
N_CHUNKS = TROWS_W * (S // CW)
HGROUPS = (CHUNK // 2) // L
NBUF = 2

_mesh = plsc.VectorSubcoreMesh(core_axis_name="c", subcore_axis_name="s")


@functools.partial(
    pl.kernel,
    out_type=jax.ShapeDtypeStruct((NUM_HEADS, S, S), jnp.float32),
    mesh=_mesh,
    scratch_types=[
        pltpu.VMEM((NUM_HEADS * NUM_TYPES,), jnp.float32),
        pltpu.VMEM((NBUF, 8, CW), jnp.int32),
        pltpu.VMEM((NBUF, 8, 8, CW), jnp.float32),
        pltpu.VMEM((2, 8, 8, CW // 2), jnp.float32),
        pltpu.SemaphoreType.DMA,
        pltpu.SemaphoreType.DMA,
        pltpu.SemaphoreType.DMA,
    ],
    compiler_params=pltpu.CompilerParams(
        needs_layout_passes=False, use_tc_tiling_on_sc=True),
)
def _edge_bias_sc(idx_hbm, tbl_hbm, out_hbm, tbl_v, idx_v, a_v, b_v, in_sem,
                  a_sem, b_sem):
    wid = lax.axis_index("s") * NC + lax.axis_index("c")
    row0 = wid * TROWS_W * 8
    cpr = S // CW

    def chunk_slices(c):
        r = row0 + (c // cpr) * 8
        col = (c % cpr) * CW
        return pl.ds(r, 8), col

    ha = pl.ds(0, 8)
    hb = pl.ds(8, 8)

    pltpu.sync_copy(tbl_hbm, tbl_v)
    r0, c0 = chunk_slices(0)
    pltpu.async_copy(idx_hbm.at[r0, pl.ds(c0, CW)], idx_v.at[0], in_sem)

    def pair_body(p, carry):
        for b in range(NBUF):
            c = p * NBUF + b
            rs, col = chunk_slices(c)
            nb = (b + 1) % NBUF

            @pl.when(c + 1 < N_CHUNKS)
            def _prefetch():
                nrs, ncol = chunk_slices(c + 1)
                pltpu.async_copy(idx_hbm.at[nrs, pl.ds(ncol, CW)],
                                 idx_v.at[nb], in_sem)

            pltpu.make_async_copy(idx_hbm.at[rs, pl.ds(col, CW)],
                                  idx_v.at[b], in_sem).wait()

            @pl.when(c >= NBUF)
            def _drain_a():
                prs, pcol = chunk_slices(c - NBUF)
                pltpu.make_async_copy(a_v.at[b],
                                      out_hbm.at[ha, prs, pl.ds(pcol, CW)],
                                      a_sem).wait()

            for j in range(2):
                jcol = j * (CW // 2)

                @pl.when(c >= 1)
                def _drain_b():
                    prs, pcol = chunk_slices(c - 1)
                    pltpu.make_async_copy(
                        b_v.at[j],
                        out_hbm.at[hb, prs, pl.ds(pcol + jcol, CW // 2)],
                        b_sem).wait()

                @plsc.parallel_loop(0, HGROUPS, unroll=2)
                def grp_body(g):
                    row = g // ((CW // 2) // L)
                    colr = (g % ((CW // 2) // L)) * L
                    idx = idx_v[b, row, pl.ds(colr + jcol, L)]
                    for h in range(8):
                        vals = plsc.load_gather(tbl_v, [idx + h * NUM_TYPES])
                        a_v[b, h, row, pl.ds(colr + jcol, L)] = vals
                    for h in range(8, NUM_HEADS):
                        vals = plsc.load_gather(tbl_v, [idx + h * NUM_TYPES])
                        b_v[j, h - 8, row, pl.ds(colr, L)] = vals

                pltpu.async_copy(
                    b_v.at[j],
                    out_hbm.at[hb, rs, pl.ds(col + jcol, CW // 2)], b_sem)

            pltpu.async_copy(a_v.at[b], out_hbm.at[ha, rs, pl.ds(col, CW)],
                             a_sem)
        return carry

    lax.fori_loop(0, N_CHUNKS // NBUF, pair_body, 0)

    rl, coll = chunk_slices(N_CHUNKS - 1)
    for j in range(2):
        pltpu.make_async_copy(
            b_v.at[j],
            out_hbm.at[hb, rl, pl.ds(coll + j * (CW // 2), CW // 2)],
            b_sem).wait()
    for b in range(NBUF):
        prs, pcol = chunk_slices(N_CHUNKS - NBUF + b)
        pltpu.make_async_copy(a_v.at[b],
                              out_hbm.at[ha, prs, pl.ds(pcol, CW)],
                              a_sem).wait()


def kernel(edge_type_matrix, edge_embedding_weight):
    idx = edge_type_matrix.astype(jnp.int32)
    tbl = edge_embedding_weight.T.reshape(-1)
    return _edge_bias_sc(idx, tbl)


# speedup vs baseline: 1.0560x; 1.0560x over previous
import functools

import jax
import jax.numpy as jnp
from jax import lax
from jax.experimental import pallas as pl
from jax.experimental.pallas import tpu as pltpu
from jax.experimental.pallas import tpu_sc as plsc

NUM_HEADS = 16
NUM_TYPES = 32
S = 2048
N = S * S

NC = 2
NS = 16
L = 16
NW = NC * NS
TROWS = S // 8
TROWS_W = TROWS // NW
CW = 512
CHUNK = 8 * CW
N_CHUNKS = TROWS_W * (S // CW)
GROUPS = CHUNK // L
HHALF = NUM_HEADS // 2
NBUF = 2

_mesh = plsc.VectorSubcoreMesh(core_axis_name="c", subcore_axis_name="s")


@functools.partial(
    pl.kernel,
    out_type=jax.ShapeDtypeStruct((NUM_HEADS, S, S), jnp.float32),
    mesh=_mesh,
    scratch_types=[
        pltpu.VMEM((NUM_HEADS * NUM_TYPES,), jnp.float32),
        pltpu.VMEM((NBUF, 8, CW), jnp.int32),
        pltpu.VMEM((2, HHALF, 8, CW), jnp.float32),
        pltpu.SemaphoreType.DMA,
        pltpu.SemaphoreType.DMA,
    ],
    compiler_params=pltpu.CompilerParams(
        needs_layout_passes=False, use_tc_tiling_on_sc=True),
)
def _edge_bias_sc(idx_hbm, tbl_hbm, out_hbm, tbl_v, idx_v, out_v, in_sem,
                  out_sem):
    wid = lax.axis_index("s") * NC + lax.axis_index("c")
    row0 = wid * TROWS_W * 8
    cpr = S // CW

    def chunk_slices(c):
        r = row0 + (c // cpr) * 8
        col = (c % cpr) * CW
        return pl.ds(r, 8), pl.ds(col, CW)

    pltpu.sync_copy(tbl_hbm, tbl_v)
    r0, c0 = chunk_slices(0)
    pltpu.async_copy(idx_hbm.at[r0, c0], idx_v.at[0], in_sem)

    def pair_body(p, carry):
        for b in range(NBUF):
            c = p * NBUF + b
            rs, cs = chunk_slices(c)
            nb = (b + 1) % NBUF

            @pl.when(c + 1 < N_CHUNKS)
            def _prefetch():
                nrs, ncs = chunk_slices(c + 1)
                pltpu.async_copy(idx_hbm.at[nrs, ncs], idx_v.at[nb], in_sem)

            pltpu.make_async_copy(idx_hbm.at[rs, cs], idx_v.at[b],
                                  in_sem).wait()

            for half in range(2):
                h0 = half * HHALF
                hs = pl.ds(h0, HHALF)

                @pl.when(c >= 1)
                def _drain():
                    prs, pcs = chunk_slices(c - 1)
                    pltpu.make_async_copy(out_v.at[half],
                                          out_hbm.at[hs, prs, pcs],
                                          out_sem).wait()

                @plsc.parallel_loop(0, GROUPS, unroll=4)
                def grp_body(g):
                    row = g // (CW // L)
                    col = (g % (CW // L)) * L
                    idx = idx_v[b, row, pl.ds(col, L)]
                    for hh in range(HHALF):
                        vals = plsc.load_gather(
                            tbl_v, [idx + (h0 + hh) * NUM_TYPES])
                        out_v[half, hh, row, pl.ds(col, L)] = vals

                pltpu.async_copy(out_v.at[half], out_hbm.at[hs, rs, cs],
                                 out_sem)
        return carry

    lax.fori_loop(0, N_CHUNKS // NBUF, pair_body, 0)
    rl, cl = chunk_slices(N_CHUNKS - 1)
    for half in range(2):
        hs = pl.ds(half * HHALF, HHALF)
        pltpu.make_async_copy(out_v.at[half], out_hbm.at[hs, rl, cl],
                              out_sem).wait()


def kernel(edge_type_matrix, edge_embedding_weight):
    idx = edge_type_matrix.astype(jnp.int32)
    tbl = edge_embedding_weight.T.reshape(-1)
    return _edge_bias_sc(idx, tbl)


# speedup vs baseline: 1.0564x; 1.0004x over previous
import functools

import jax
import jax.numpy as jnp
from jax import lax
from jax.experimental import pallas as pl
from jax.experimental.pallas import tpu as pltpu
from jax.experimental.pallas import tpu_sc as plsc

NUM_HEADS = 16
NUM_TYPES = 32
S = 2048
N = S * S

NC = 2
NS = 16
L = 16
NW = NC * NS
TROWS = S // 8
TROWS_W = TROWS // NW
CW = 512
CHUNK = 8 * CW
N_CHUNKS = TROWS_W * (S // CW)
GROUPS = CHUNK // L
HHALF = NUM_HEADS // 2
NBUF = 2

_mesh = plsc.VectorSubcoreMesh(core_axis_name="c", subcore_axis_name="s")


@functools.partial(
    pl.kernel,
    out_type=jax.ShapeDtypeStruct((NUM_HEADS, S, S), jnp.float32),
    mesh=_mesh,
    scratch_types=[
        pltpu.VMEM((NUM_HEADS * NUM_TYPES,), jnp.float32),
        pltpu.VMEM((NUM_TYPES, NUM_HEADS), jnp.float32),
        pltpu.VMEM((NBUF, 8, CW), jnp.int32),
        pltpu.VMEM((2, HHALF, 8, CW), jnp.float32),
        pltpu.SemaphoreType.DMA,
        pltpu.SemaphoreType.DMA,
    ],
    compiler_params=pltpu.CompilerParams(
        needs_layout_passes=False, use_tc_tiling_on_sc=True),
)
def _edge_bias_sc(idx_hbm, tbl_hbm, out_hbm, tbl_v, tbl2_v, idx_v, out_v, in_sem,
                  out_sem):
    wid = lax.axis_index("s") * NC + lax.axis_index("c")
    row0 = wid * TROWS_W * 8
    cpr = S // CW

    def chunk_slices(c):
        r = row0 + (c // cpr) * 8
        col = (c % cpr) * CW
        return pl.ds(r, 8), pl.ds(col, CW)

    r0, c0 = chunk_slices(0)
    pltpu.async_copy(idx_hbm.at[r0, c0], idx_v.at[0], in_sem)
    pltpu.sync_copy(tbl_hbm, tbl2_v)
    hofs = lax.iota(jnp.int32, L) * NUM_TYPES
    for t in range(NUM_TYPES):
        plsc.store_scatter(tbl_v, [hofs + t], tbl2_v[t, :])

    def pair_body(p, carry):
        for b in range(NBUF):
            c = p * NBUF + b
            rs, cs = chunk_slices(c)
            nb = (b + 1) % NBUF

            @pl.when(c + 1 < N_CHUNKS)
            def _prefetch():
                nrs, ncs = chunk_slices(c + 1)
                pltpu.async_copy(idx_hbm.at[nrs, ncs], idx_v.at[nb], in_sem)

            pltpu.make_async_copy(idx_hbm.at[rs, cs], idx_v.at[b],
                                  in_sem).wait()

            for half in range(2):
                h0 = half * HHALF
                hs = pl.ds(h0, HHALF)

                @pl.when(c >= 1)
                def _drain():
                    prs, pcs = chunk_slices(c - 1)
                    pltpu.make_async_copy(out_v.at[half],
                                          out_hbm.at[hs, prs, pcs],
                                          out_sem).wait()

                @plsc.parallel_loop(0, GROUPS, unroll=4)
                def grp_body(g):
                    row = g // (CW // L)
                    col = (g % (CW // L)) * L
                    idx = idx_v[b, row, pl.ds(col, L)]
                    for hh in range(HHALF):
                        vals = plsc.load_gather(
                            tbl_v, [idx + (h0 + hh) * NUM_TYPES])
                        out_v[half, hh, row, pl.ds(col, L)] = vals

                pltpu.async_copy(out_v.at[half], out_hbm.at[hs, rs, cs],
                                 out_sem)
        return carry

    lax.fori_loop(0, N_CHUNKS // NBUF, pair_body, 0)
    rl, cl = chunk_slices(N_CHUNKS - 1)
    for half in range(2):
        hs = pl.ds(half * HHALF, HHALF)
        pltpu.make_async_copy(out_v.at[half], out_hbm.at[hs, rl, cl],
                              out_sem).wait()


def kernel(edge_type_matrix, edge_embedding_weight):
    idx = edge_type_matrix.astype(jnp.int32)
    return _edge_bias_sc(idx, edge_embedding_weight)
